# Initial kernel scaffold; baseline (speedup 1.0000x reference)
import jax
import jax.numpy as jnp
from jax.experimental import pallas as pl


def kernel(phi, succ_feats, W):
    raise NotImplementedError("write your pallas kernel here")


# speedup vs baseline: 13.8728x; 13.8728x over previous
import functools

import jax
import jax.numpy as jnp
from jax import lax
from jax.experimental import pallas as pl
from jax.experimental.pallas import tpu as pltpu
from jax.experimental.pallas import tpu_sc as plsc

_B = 4096
_ROWW = 20
_NC = 2
_NS = 16
_L = 16
_NW = _NC * _NS
_RPW = _B // _NW
_CHUNKS = _RPW // _L


def _sc_body(phi_hbm, w_hbm, out_hbm, pv, wv, ov):
    wid = lax.axis_index("s") * _NC + lax.axis_index("c")
    pltpu.sync_copy(phi_hbm.at[pl.ds(wid * (_RPW * _ROWW), _RPW * _ROWW)], pv)
    pltpu.sync_copy(w_hbm, wv)
    row_off = lax.iota(jnp.int32, _L) * _ROWW
    pair_off = lax.iota(jnp.int32, _L) * 2
    for chunk in range(_CHUNKS):
        base = chunk * _L * _ROWW
        d = jnp.zeros((_L,), jnp.float32)
        for c in range(6):
            left = plsc.load_gather(pv, [row_off + (base + c)])
            right = plsc.load_gather(pv, [row_off + (base + 10 + c)])
            d = d + (left - right) * wv[c]
        s = 1.0 / (1.0 + jnp.exp(-d))
        oidx = pair_off + chunk * _L * 2
        plsc.store_scatter(ov, [oidx], s)
        plsc.store_scatter(ov, [oidx + 1], 1.0 - s)
    pltpu.sync_copy(ov, out_hbm.at[pl.ds(wid * (_RPW * 2), _RPW * 2)])


@functools.lru_cache(maxsize=1)
def _sc_call():
    mesh = plsc.VectorSubcoreMesh(core_axis_name="c", subcore_axis_name="s")
    return pl.kernel(
        _sc_body,
        mesh=mesh,
        compiler_params=pltpu.CompilerParams(needs_layout_passes=False),
        out_type=jax.ShapeDtypeStruct((_B * 2,), jnp.float32),
        scratch_types=[
            pltpu.VMEM((_RPW * _ROWW,), jnp.float32),
            pltpu.VMEM((6, _L), jnp.float32),
            pltpu.VMEM((_RPW * 2,), jnp.float32),
        ],
    )


def kernel(phi, succ_feats, W):
    del succ_feats
    phi32 = phi.astype(jnp.float32).reshape(_B * _ROWW)
    wsp = jnp.broadcast_to(W.astype(jnp.float32).reshape(6, 1), (6, _L))
    out = _sc_call()(phi32, wsp)
    return out.reshape(_B, 2, 1).astype(jnp.float64)
